# baseline (speedup 1.0000x reference)
import functools

import jax
import jax.numpy as jnp
import numpy as np
from jax import lax
from jax.experimental import pallas as pl
from jax.experimental.pallas import tpu as pltpu
from jax.experimental.pallas import tpu_sc as plsc

N = 10000
E = 320000
B = 64
IN_DIM = 128
HID = 128
HEADS = 8
CPH = 16
EMB = 256
BN_SCALE = float(1.0 / np.sqrt(1.0 + 1e-5))
NEG_BIG = -3.4e38

NC, NS, LANES = 2, 16, 16
NW = NC * NS
CHUNK = 64
NCHUNK = 164
EW = CHUNK * NCHUNK
EP = EW * NW
NPAD = 10112
RPS = NPAD // NS
ACC_W = HID + LANES


def _ln(x, eps=1e-5):
    m = jnp.mean(x, axis=-1, keepdims=True)
    v = jnp.mean((x - m) ** 2, axis=-1, keepdims=True)
    return (x - m) / jnp.sqrt(v + eps)


def _leaky(x):
    return jnp.maximum(x, 0.2 * x)


def _dot(a, b):
    return jax.lax.dot_general(a, b, (((1,), (0,)), ((), ())),
                               preferred_element_type=jnp.float32)


NBLK = 5
BLK = N // NBLK


def _prep(h, W_ref, asel_ref, adsel_ref, xp_ref, asn_ref, adn_ref, gmax_ref,
          mas_s, mad_s):
    i = pl.program_id(0)
    xp = _dot(h, W_ref[...])
    xp_ref[...] = xp

    def headsum(vec):
        prod = xp * vec
        return jnp.concatenate(
            [jnp.sum(prod[:, 16 * hh:16 * hh + 16], axis=1, keepdims=True)
             for hh in range(HEADS)], axis=1)

    asn = headsum(asel_ref[...])
    adn = headsum(adsel_ref[...])
    asn_ref[...] = asn
    adn_ref[...] = adn

    @pl.when(i == 0)
    def _init():
        mas_s[...] = jnp.full_like(mas_s, NEG_BIG)
        mad_s[...] = jnp.full_like(mad_s, NEG_BIG)

    mas_s[...] = jnp.maximum(mas_s[...], jnp.max(asn, axis=0, keepdims=True))
    mad_s[...] = jnp.maximum(mad_s[...], jnp.max(adn, axis=0, keepdims=True))

    @pl.when(i == NBLK - 1)
    def _final():
        gmax_ref[...] = _leaky(mas_s[...] + mad_s[...])


def _tc_in_body(x_ref, inW_ref, inb_ref, W_ref, asel_ref, adsel_ref,
                h_ref, xp_ref, asn_ref, adn_ref, gmax_ref, mas_s, mad_s):
    h = jnp.maximum(BN_SCALE * (_dot(x_ref[...], inW_ref[...]) + inb_ref[...]), 0.0)
    h_ref[...] = h
    _prep(h, W_ref, asel_ref, adsel_ref, xp_ref, asn_ref, adn_ref, gmax_ref,
          mas_s, mad_s)


def _acc_combine(acc_ref, sexp_ref):
    s = acc_ref[0, :, :HID] + acc_ref[1, :, :HID]
    den8 = (acc_ref[0, :, HID:HID + HEADS] + acc_ref[1, :, HID:HID + HEADS])
    den = _dot(den8, sexp_ref[...])
    return s / (den + 1e-16)


def _tc_mid_body(acc_ref, sexp_ref, b_ref, hr_ref, W_ref, asel_ref,
                 adsel_ref, h_ref, xp_ref, asn_ref, adn_ref, gmax_ref,
                 mas_s, mad_s):
    g = _acc_combine(acc_ref, sexp_ref) + b_ref[...]
    y = jnp.maximum(_ln(BN_SCALE * g), 0.0)
    h = y + hr_ref[...]
    h_ref[...] = h
    _prep(h, W_ref, asel_ref, adsel_ref, xp_ref, asn_ref, adn_ref, gmax_ref,
          mas_s, mad_s)


def _tc_last_body(acc_ref, sexp_ref, mavg_ref, b_ref, h_ref):
    out = _acc_combine(acc_ref, sexp_ref)
    g = _dot(out, mavg_ref[...]) + b_ref[...]
    h_ref[...] = jnp.maximum(_ln(BN_SCALE * g), 0.0)


def _tc_head_body(h_ref, batch_ref, dop_ref, poolW_ref, poolb_ref,
                  embW_ref, embb_ref, p0W_ref, p0b_ref, p1W_ref, p1b_ref,
                  p2W_ref, p2b_ref, headW_ref, headb_ref,
                  lat_ref, emb_ref, sum_s, cnt_s, hx_s, *, nblocks):
    i = pl.program_id(0)

    @pl.when(i == 0)
    def _init():
        sum_s[...] = jnp.zeros_like(sum_s)
        cnt_s[...] = jnp.zeros_like(cnt_s)
        hx_s[...] = jnp.zeros_like(hx_s)

    hb = h_ref[...]
    bb = batch_ref[...]
    rows = hb.shape[0]
    mask = bb == lax.broadcasted_iota(jnp.int32, (rows, B), 1)
    mf = mask.astype(jnp.float32)
    sum_s[...] += lax.dot_general(mf, hb, (((0,), (0,)), ((), ())),
                                  preferred_element_type=jnp.float32)
    cnt_s[...] += lax.dot_general(mf, jnp.ones((rows, CPH), jnp.float32),
                                  (((0,), (0,)), ((), ())),
                                  preferred_element_type=jnp.float32)
    for b in range(B):
        mx = jnp.max(mf[:, b:b + 1] * hb, axis=0, keepdims=True)
        hx_s[b:b + 1, :] = jnp.maximum(hx_s[b:b + 1, :], mx)

    @pl.when(i == nblocks - 1)
    def _final():
        hm = sum_s[...] / jnp.maximum(cnt_s[...], 1.0)
        hx = hx_s[...]
        hp = jnp.concatenate([hm, hx], axis=1)
        hp = jnp.maximum(BN_SCALE * (_dot(hp, poolW_ref[...]) + poolb_ref[...]), 0.0)
        emb = jnp.maximum(_ln(BN_SCALE * (_dot(hp, embW_ref[...]) + embb_ref[...])), 0.0)
        emb_ref[...] = emb
        z = jnp.concatenate([emb, dop_ref[...], jnp.zeros((B, 7), jnp.float32)], axis=1)
        for Wr, br in ((p0W_ref, p0b_ref), (p1W_ref, p1b_ref), (p2W_ref, p2b_ref)):
            z = jnp.maximum(_ln(BN_SCALE * (_dot(z, Wr[...]) + br[...])), 0.0)
        lat_ref[...] = _dot(z, headW_ref[...]) + headb_ref[...]


def _whole(x):
    return pl.BlockSpec(x.shape, lambda i=0: tuple(0 for _ in x.shape))


_PREP_OUTS = (
    jax.ShapeDtypeStruct((N, HID), jnp.float32),
    jax.ShapeDtypeStruct((N, HID), jnp.float32),
    jax.ShapeDtypeStruct((N, HEADS), jnp.float32),
    jax.ShapeDtypeStruct((N, HEADS), jnp.float32),
    jax.ShapeDtypeStruct((1, HEADS), jnp.float32),
)

_PREP_OUT_SPECS = (
    pl.BlockSpec((BLK, HID), lambda i: (i, 0)),
    pl.BlockSpec((BLK, HID), lambda i: (i, 0)),
    pl.BlockSpec((BLK, HEADS), lambda i: (i, 0)),
    pl.BlockSpec((BLK, HEADS), lambda i: (i, 0)),
    pl.BlockSpec((1, HEADS), lambda i: (0, 0)),
)

_PREP_SCRATCH = [
    pltpu.VMEM((1, HEADS), jnp.float32),
    pltpu.VMEM((1, HEADS), jnp.float32),
]

_ACC_SPEC = pl.BlockSpec((NC, BLK, ACC_W), lambda i: (0, i, 0))


def _tc_in(x, inW, inb, W, asel, adsel):
    return pl.pallas_call(
        _tc_in_body,
        grid=(NBLK,),
        in_specs=[pl.BlockSpec((BLK, IN_DIM), lambda i: (i, 0)),
                  _whole(inW), _whole(inb), _whole(W), _whole(asel),
                  _whole(adsel)],
        out_specs=_PREP_OUT_SPECS,
        out_shape=_PREP_OUTS,
        scratch_shapes=_PREP_SCRATCH,
    )(x, inW, inb, W, asel, adsel)


def _tc_mid(acc, sexp, b, hr, W, asel, adsel):
    return pl.pallas_call(
        _tc_mid_body,
        grid=(NBLK,),
        in_specs=[_ACC_SPEC, _whole(sexp), _whole(b),
                  pl.BlockSpec((BLK, HID), lambda i: (i, 0)),
                  _whole(W), _whole(asel), _whole(adsel)],
        out_specs=_PREP_OUT_SPECS,
        out_shape=_PREP_OUTS,
        scratch_shapes=_PREP_SCRATCH,
    )(acc, sexp, b, hr, W, asel, adsel)


def _tc_last(acc, sexp, mavg, b):
    return pl.pallas_call(
        _tc_last_body,
        grid=(NBLK,),
        in_specs=[_ACC_SPEC, _whole(sexp), _whole(mavg), _whole(b)],
        out_specs=pl.BlockSpec((BLK, CPH), lambda i: (i, 0)),
        out_shape=jax.ShapeDtypeStruct((N, CPH), jnp.float32),
    )(acc, sexp, mavg, b)


def _tc_head(h2, batch2, dop2, pw, pb, ew, eb, p0W, p0b, p1W, p1b, p2W, p2b, hW, hb):
    nblocks = 25
    rows = N // nblocks
    weights = [pw, pb, ew, eb, p0W, p0b, p1W, p1b, p2W, p2b, hW, hb]
    return pl.pallas_call(
        functools.partial(_tc_head_body, nblocks=nblocks),
        grid=(nblocks,),
        in_specs=[
            pl.BlockSpec((rows, CPH), lambda i: (i, 0)),
            pl.BlockSpec((rows, 1), lambda i: (i, 0)),
            _whole(dop2),
        ] + [_whole(w) for w in weights],
        out_specs=(
            pl.BlockSpec((B, 1), lambda i: (0, 0)),
            pl.BlockSpec((B, EMB), lambda i: (0, 0)),
        ),
        out_shape=(
            jax.ShapeDtypeStruct((B, 1), jnp.float32),
            jax.ShapeDtypeStruct((B, EMB), jnp.float32),
        ),
        scratch_shapes=[
            pltpu.VMEM((B, CPH), jnp.float32),
            pltpu.VMEM((B, CPH), jnp.float32),
            pltpu.VMEM((B, CPH), jnp.float32),
        ],
    )(h2, batch2, dop2, *weights)


def _vgather(v, idx):
    dn = lax.GatherDimensionNumbers(
        offset_dims=(), collapsed_slice_dims=(0,), start_index_map=(0,))
    return lax.gather(v, idx[:, None], dn, (1,),
                      mode=lax.GatherScatterMode.PROMISE_IN_BOUNDS)


def _sc_edge_body(src_hbm, dst_hbm, atab_hbm, xp_hbm, gmax_hbm, zeros_hbm,
                  out_hbm,
                  srcv0, srcv1, dstv0, dstv1, dstv20, dstv21,
                  asg0, asg1, adg0, adg1, xpg0, xpg1, msg0, msg1, gbuf, acc,
                  sem_si0, sem_si1, sem_di0, sem_di1, sem_as0, sem_as1,
                  sem_ad0, sem_ad1, sem_xp0, sem_xp1, sem_sc0, sem_sc1):
    srcv = (srcv0, srcv1)
    dstv = (dstv0, dstv1)
    dstv2 = (dstv20, dstv21)
    asg = (asg0, asg1)
    adg = (adg0, adg1)
    xpg = (xpg0, xpg1)
    msg = (msg0, msg1)
    sem_si = (sem_si0, sem_si1)
    sem_di = (sem_di0, sem_di1)
    sem_as = (sem_as0, sem_as1)
    sem_ad = (sem_ad0, sem_ad1)
    sem_xp = (sem_xp0, sem_xp1)
    sem_sc = (sem_sc0, sem_sc1)
    c = lax.axis_index("c")
    s = lax.axis_index("s")
    wid = s * NC + c
    pltpu.sync_copy(zeros_hbm.at[pl.ds(s * RPS, RPS)],
                    acc.at[pl.ds(s * RPS, RPS)])
    pltpu.sync_copy(gmax_hbm, gbuf)
    plsc.subcore_barrier()

    lane = lax.iota(jnp.int32, LANES)
    mask8 = lane < HEADS
    shift_idx = jnp.minimum(lane + HEADS, LANES - 1)
    base0 = wid * EW
    gmaxv = gbuf[...]

    def issue_idx(b, ci):
        base = base0 + ci * CHUNK
        pltpu.async_copy(src_hbm.at[pl.ds(base, CHUNK)], srcv[b], sem_si[b])
        pltpu.async_copy(dst_hbm.at[pl.ds(base, CHUNK)], dstv[b], sem_di[b])

    def wait_idx(b):
        pltpu.make_async_copy(src_hbm.at[pl.ds(0, CHUNK)], srcv[b], sem_si[b]).wait()
        pltpu.make_async_copy(dst_hbm.at[pl.ds(0, CHUNK)], dstv[b], sem_di[b]).wait()

    def issue_gathers(b):
        pltpu.async_copy(atab_hbm.at[srcv[b]], asg[b], sem_as[b])
        pltpu.async_copy(atab_hbm.at[dstv[b]], adg[b], sem_ad[b])
        pltpu.async_copy(xp_hbm.at[srcv[b]], xpg[b], sem_xp[b])

    def wait_gathers(b):
        pltpu.make_async_copy(atab_hbm.at[srcv[b]], asg[b], sem_as[b]).wait()
        pltpu.make_async_copy(atab_hbm.at[dstv[b]], adg[b], sem_ad[b]).wait()
        pltpu.make_async_copy(xp_hbm.at[srcv[b]], xpg[b], sem_xp[b]).wait()

    def wait_scatter(b):
        pltpu.make_async_copy(msg[b], acc.at[dstv2[b]], sem_sc[b]).wait()

    def compute(b):
        asg_b, adg_b, xpg_b, msg_b = asg[b], adg[b], xpg[b], msg[b]

        def edge_body(e, carry2):
            a1 = asg_b[e, :]
            a2 = _vgather(adg_b[e, :], shift_idx)
            al = a1 + a2
            al = jnp.maximum(al, 0.2 * al)
            ea = jnp.exp(al - gmaxv)
            ea = jnp.where(mask8, ea, 0.0)
            msg_b[e, pl.ds(HID, LANES)] = ea
            for hh in range(HEADS):
                sp = _vgather(ea, lane * 0 + hh)
                msg_b[e, pl.ds(hh * LANES, LANES)] = (
                    xpg_b[e, pl.ds(hh * LANES, LANES)] * sp)
            return carry2

        lax.fori_loop(0, CHUNK, edge_body, 0)

    def process(b, ci, first):
        wait_gathers(b)
        if not first:
            wait_scatter(b)
        for k in range(CHUNK // LANES):
            dstv2[b][pl.ds(k * LANES, LANES)] = dstv[b][pl.ds(k * LANES, LANES)]
        cn = jnp.minimum(ci + 2, NCHUNK - 1)
        issue_idx(b, cn)
        compute(b)
        pltpu.async_copy(msg[b], acc.at[dstv2[b]], sem_sc[b], add=True)
        wait_idx(b)
        issue_gathers(b)

    for b in (0, 1):
        issue_idx(b, b)
        wait_idx(b)
        issue_gathers(b)
    for b in (0, 1):
        process(b, jnp.int32(b), True)

    def steady(g, carry):
        process(0, 2 * g, False)
        process(1, 2 * g + 1, False)
        return carry

    lax.fori_loop(1, NCHUNK // 2, steady, 0)
    for b in (0, 1):
        wait_gathers(b)
        wait_scatter(b)
    plsc.subcore_barrier()
    pltpu.sync_copy(acc.at[pl.ds(s * RPS, RPS)],
                    out_hbm.at[c, pl.ds(s * RPS, RPS)])


_sc_edge = pl.kernel(
    _sc_edge_body,
    out_type=jax.ShapeDtypeStruct((NC, NPAD, ACC_W), jnp.float32),
    mesh=plsc.VectorSubcoreMesh(core_axis_name="c", subcore_axis_name="s"),
    compiler_params=pltpu.CompilerParams(use_tc_tiling_on_sc=False),
    scratch_types=(
        [pltpu.VMEM((CHUNK,), jnp.int32)] * 6
        + [pltpu.VMEM((CHUNK, LANES), jnp.float32)] * 4
        + [pltpu.VMEM((CHUNK, HID), jnp.float32)] * 2
        + [pltpu.VMEM((CHUNK, ACC_W), jnp.float32)] * 2
        + [pltpu.VMEM((LANES,), jnp.float32)]
        + [pltpu.VMEM_SHARED((NPAD, ACC_W), jnp.float32)]
        + [pltpu.SemaphoreType.DMA] * 12
    ),
)


def _edge_phase(xp, asn, adn, gmax, src_p, dst_p, zeros_tab):
    atab = jnp.concatenate([asn, adn], axis=1)
    atab = jnp.pad(atab, ((0, NPAD - N), (0, 0)), constant_values=-1e30)
    xp_p = jnp.pad(xp, ((0, NPAD - N), (0, 0)))
    gmax16 = jnp.pad(gmax, ((0, 0), (0, LANES - HEADS))).reshape(LANES)
    return _sc_edge(src_p, dst_p, atab, xp_p, gmax16, zeros_tab)


def kernel(x, edge_index, batch, dop_levels, params):
    p = params
    loops = jnp.arange(N, dtype=edge_index.dtype)
    pad = jnp.full((EP - E - N,), N, dtype=edge_index.dtype)
    src_p = jnp.concatenate([edge_index[0], loops, pad])
    dst_p = jnp.concatenate([edge_index[1], loops, pad])
    zeros_tab = jnp.zeros((NPAD, ACC_W), jnp.float32)

    def asel(a):
        return a.reshape(1, HEADS * CPH)

    sexp = jax.scipy.linalg.block_diag(*([jnp.ones((1, CPH), jnp.float32)] * HEADS))
    mavg = jnp.tile(jnp.eye(CPH, dtype=jnp.float32), (HEADS, 1)) / HEADS

    h, xp, asn, adn, gmax = _tc_in(
        x, p['in_W'], p['in_b'].reshape(1, -1), p['g0_W'],
        asel(p['g0_as']), asel(p['g0_ad']))

    for i in (0, 1, 2):
        acc = _edge_phase(xp, asn, adn, gmax, src_p, dst_p, zeros_tab)
        if i < 2:
            j = i + 1
            h, xp, asn, adn, gmax = _tc_mid(
                acc, sexp, p['g%d_b' % i].reshape(1, -1), h, p['g%d_W' % j],
                asel(p['g%d_as' % j]), asel(p['g%d_ad' % j]))
        else:
            h2 = _tc_last(acc, sexp, mavg, p['g2_b'].reshape(1, -1))

    lat, emb = _tc_head(
        h2, batch.reshape(N, 1), dop_levels.reshape(B, 1),
        p['pool_W'], p['pool_b'].reshape(1, -1),
        p['emb_W'], p['emb_b'].reshape(1, -1),
        jnp.pad(p['p0_W'], ((0, 7), (0, 0))), p['p0_b'].reshape(1, -1),
        p['p1_W'], p['p1_b'].reshape(1, -1),
        p['p2_W'], p['p2_b'].reshape(1, -1),
        p['head_W'], p['head_b'].reshape(1, -1))
    return lat, emb


# speedup vs baseline: 1.2406x; 1.2406x over previous
import functools

import jax
import jax.numpy as jnp
import numpy as np
from jax import lax
from jax.experimental import pallas as pl
from jax.experimental.pallas import tpu as pltpu
from jax.experimental.pallas import tpu_sc as plsc

N = 10000
E = 320000
B = 64
IN_DIM = 128
HID = 128
HEADS = 8
CPH = 16
EMB = 256
BN_SCALE = float(1.0 / np.sqrt(1.0 + 1e-5))
NEG_BIG = -3.4e38

NC, NS, LANES = 2, 16, 16
NW = NC * NS
CHUNK = 64
NCHUNK = 164
EW = CHUNK * NCHUNK
EP = EW * NW
NPAD = 10112
RPS = NPAD // NS
ACC_W = HID + LANES


def _ln(x, eps=1e-5):
    m = jnp.mean(x, axis=-1, keepdims=True)
    v = jnp.mean((x - m) ** 2, axis=-1, keepdims=True)
    return (x - m) / jnp.sqrt(v + eps)


def _leaky(x):
    return jnp.maximum(x, 0.2 * x)


def _dot(a, b):
    return jax.lax.dot_general(a, b, (((1,), (0,)), ((), ())),
                               preferred_element_type=jnp.float32)


NBLK = 5
BLK = N // NBLK


def _prep(h, W_ref, asel_ref, adsel_ref, xp_ref, asn_ref, adn_ref, gmax_ref,
          mas_s, mad_s):
    i = pl.program_id(0)
    xp = _dot(h, W_ref[...])
    xp_ref[...] = xp
    asn = _dot(xp, asel_ref[...])
    adn = _dot(xp, adsel_ref[...])
    asn_ref[...] = asn
    adn_ref[...] = adn

    @pl.when(i == 0)
    def _init():
        mas_s[...] = jnp.full_like(mas_s, NEG_BIG)
        mad_s[...] = jnp.full_like(mad_s, NEG_BIG)

    mas_s[...] = jnp.maximum(mas_s[...], jnp.max(asn, axis=0, keepdims=True))
    mad_s[...] = jnp.maximum(mad_s[...], jnp.max(adn, axis=0, keepdims=True))

    @pl.when(i == NBLK - 1)
    def _final():
        gmax_ref[...] = _leaky(mas_s[...] + mad_s[...])


def _tc_in_body(x_ref, inW_ref, inb_ref, W_ref, asel_ref, adsel_ref,
                h_ref, xp_ref, asn_ref, adn_ref, gmax_ref, mas_s, mad_s):
    h = jnp.maximum(BN_SCALE * (_dot(x_ref[...], inW_ref[...]) + inb_ref[...]), 0.0)
    h_ref[...] = h
    _prep(h, W_ref, asel_ref, adsel_ref, xp_ref, asn_ref, adn_ref, gmax_ref,
          mas_s, mad_s)


def _acc_combine(acc_ref, sexp_ref):
    s = acc_ref[0, :, :HID] + acc_ref[1, :, :HID]
    den8 = (acc_ref[0, :, HID:HID + HEADS] + acc_ref[1, :, HID:HID + HEADS])
    den = _dot(den8, sexp_ref[...])
    return s / (den + 1e-16)


def _tc_mid_body(acc_ref, sexp_ref, b_ref, hr_ref, W_ref, asel_ref,
                 adsel_ref, h_ref, xp_ref, asn_ref, adn_ref, gmax_ref,
                 mas_s, mad_s):
    g = _acc_combine(acc_ref, sexp_ref) + b_ref[...]
    y = jnp.maximum(_ln(BN_SCALE * g), 0.0)
    h = y + hr_ref[...]
    h_ref[...] = h
    _prep(h, W_ref, asel_ref, adsel_ref, xp_ref, asn_ref, adn_ref, gmax_ref,
          mas_s, mad_s)


def _tc_last_body(acc_ref, sexp_ref, mavg_ref, b_ref, h_ref):
    out = _acc_combine(acc_ref, sexp_ref)
    g = _dot(out, mavg_ref[...]) + b_ref[...]
    h_ref[...] = jnp.maximum(_ln(BN_SCALE * g), 0.0)


def _tc_head_body(h_ref, batch_ref, dop_ref, poolW_ref, poolb_ref,
                  embW_ref, embb_ref, p0W_ref, p0b_ref, p1W_ref, p1b_ref,
                  p2W_ref, p2b_ref, headW_ref, headb_ref,
                  lat_ref, emb_ref, sum_s, cnt_s, hx_s, *, nblocks):
    i = pl.program_id(0)

    @pl.when(i == 0)
    def _init():
        sum_s[...] = jnp.zeros_like(sum_s)
        cnt_s[...] = jnp.zeros_like(cnt_s)
        hx_s[...] = jnp.zeros_like(hx_s)

    hb = h_ref[...]
    bb = batch_ref[...]
    rows = hb.shape[0]
    mask = bb == lax.broadcasted_iota(jnp.int32, (rows, B), 1)
    mf = mask.astype(jnp.float32)
    sum_s[...] += lax.dot_general(mf, hb, (((0,), (0,)), ((), ())),
                                  preferred_element_type=jnp.float32)
    cnt_s[...] += lax.dot_general(mf, jnp.ones((rows, CPH), jnp.float32),
                                  (((0,), (0,)), ((), ())),
                                  preferred_element_type=jnp.float32)
    for b in range(B):
        mx = jnp.max(mf[:, b:b + 1] * hb, axis=0, keepdims=True)
        hx_s[b:b + 1, :] = jnp.maximum(hx_s[b:b + 1, :], mx)

    @pl.when(i == nblocks - 1)
    def _final():
        hm = sum_s[...] / jnp.maximum(cnt_s[...], 1.0)
        hx = hx_s[...]
        hp = jnp.concatenate([hm, hx], axis=1)
        hp = jnp.maximum(BN_SCALE * (_dot(hp, poolW_ref[...]) + poolb_ref[...]), 0.0)
        emb = jnp.maximum(_ln(BN_SCALE * (_dot(hp, embW_ref[...]) + embb_ref[...])), 0.0)
        emb_ref[...] = emb
        z = jnp.concatenate([emb, dop_ref[...], jnp.zeros((B, 7), jnp.float32)], axis=1)
        for Wr, br in ((p0W_ref, p0b_ref), (p1W_ref, p1b_ref), (p2W_ref, p2b_ref)):
            z = jnp.maximum(_ln(BN_SCALE * (_dot(z, Wr[...]) + br[...])), 0.0)
        lat_ref[...] = _dot(z, headW_ref[...]) + headb_ref[...]


def _whole(x):
    return pl.BlockSpec(x.shape, lambda i=0: tuple(0 for _ in x.shape))


_PREP_OUTS = (
    jax.ShapeDtypeStruct((N, HID), jnp.float32),
    jax.ShapeDtypeStruct((N, HID), jnp.float32),
    jax.ShapeDtypeStruct((N, HEADS), jnp.float32),
    jax.ShapeDtypeStruct((N, HEADS), jnp.float32),
    jax.ShapeDtypeStruct((1, HEADS), jnp.float32),
)

_PREP_OUT_SPECS = (
    pl.BlockSpec((BLK, HID), lambda i: (i, 0)),
    pl.BlockSpec((BLK, HID), lambda i: (i, 0)),
    pl.BlockSpec((BLK, HEADS), lambda i: (i, 0)),
    pl.BlockSpec((BLK, HEADS), lambda i: (i, 0)),
    pl.BlockSpec((1, HEADS), lambda i: (0, 0)),
)

_PREP_SCRATCH = [
    pltpu.VMEM((1, HEADS), jnp.float32),
    pltpu.VMEM((1, HEADS), jnp.float32),
]

_ACC_SPEC = pl.BlockSpec((NC, BLK, ACC_W), lambda i: (0, i, 0))


def _tc_in(x, inW, inb, W, asel, adsel):
    return pl.pallas_call(
        _tc_in_body,
        grid=(NBLK,),
        in_specs=[pl.BlockSpec((BLK, IN_DIM), lambda i: (i, 0)),
                  _whole(inW), _whole(inb), _whole(W), _whole(asel),
                  _whole(adsel)],
        out_specs=_PREP_OUT_SPECS,
        out_shape=_PREP_OUTS,
        scratch_shapes=_PREP_SCRATCH,
    )(x, inW, inb, W, asel, adsel)


def _tc_mid(acc, sexp, b, hr, W, asel, adsel):
    return pl.pallas_call(
        _tc_mid_body,
        grid=(NBLK,),
        in_specs=[_ACC_SPEC, _whole(sexp), _whole(b),
                  pl.BlockSpec((BLK, HID), lambda i: (i, 0)),
                  _whole(W), _whole(asel), _whole(adsel)],
        out_specs=_PREP_OUT_SPECS,
        out_shape=_PREP_OUTS,
        scratch_shapes=_PREP_SCRATCH,
    )(acc, sexp, b, hr, W, asel, adsel)


def _tc_last(acc, sexp, mavg, b):
    return pl.pallas_call(
        _tc_last_body,
        grid=(NBLK,),
        in_specs=[_ACC_SPEC, _whole(sexp), _whole(mavg), _whole(b)],
        out_specs=pl.BlockSpec((BLK, CPH), lambda i: (i, 0)),
        out_shape=jax.ShapeDtypeStruct((N, CPH), jnp.float32),
    )(acc, sexp, mavg, b)


def _tc_head(h2, batch2, dop2, pw, pb, ew, eb, p0W, p0b, p1W, p1b, p2W, p2b, hW, hb):
    nblocks = 25
    rows = N // nblocks
    weights = [pw, pb, ew, eb, p0W, p0b, p1W, p1b, p2W, p2b, hW, hb]
    return pl.pallas_call(
        functools.partial(_tc_head_body, nblocks=nblocks),
        grid=(nblocks,),
        in_specs=[
            pl.BlockSpec((rows, CPH), lambda i: (i, 0)),
            pl.BlockSpec((rows, 1), lambda i: (i, 0)),
            _whole(dop2),
        ] + [_whole(w) for w in weights],
        out_specs=(
            pl.BlockSpec((B, 1), lambda i: (0, 0)),
            pl.BlockSpec((B, EMB), lambda i: (0, 0)),
        ),
        out_shape=(
            jax.ShapeDtypeStruct((B, 1), jnp.float32),
            jax.ShapeDtypeStruct((B, EMB), jnp.float32),
        ),
        scratch_shapes=[
            pltpu.VMEM((B, CPH), jnp.float32),
            pltpu.VMEM((B, CPH), jnp.float32),
            pltpu.VMEM((B, CPH), jnp.float32),
        ],
    )(h2, batch2, dop2, *weights)


def _vgather(v, idx):
    dn = lax.GatherDimensionNumbers(
        offset_dims=(), collapsed_slice_dims=(0,), start_index_map=(0,))
    return lax.gather(v, idx[:, None], dn, (1,),
                      mode=lax.GatherScatterMode.PROMISE_IN_BOUNDS)


def _sc_edge_body(src_hbm, dst_hbm, atab_hbm, xp_hbm, gmax_hbm, zeros_hbm,
                  out_hbm,
                  srcv0, srcv1, dstv0, dstv1, dstv20, dstv21,
                  asg0, asg1, adg0, adg1, xpg0, xpg1, msg0, msg1, gbuf, acc,
                  sem_si0, sem_si1, sem_di0, sem_di1, sem_as0, sem_as1,
                  sem_ad0, sem_ad1, sem_xp0, sem_xp1, sem_sc0, sem_sc1):
    srcv = (srcv0, srcv1)
    dstv = (dstv0, dstv1)
    dstv2 = (dstv20, dstv21)
    asg = (asg0, asg1)
    adg = (adg0, adg1)
    xpg = (xpg0, xpg1)
    msg = (msg0, msg1)
    sem_si = (sem_si0, sem_si1)
    sem_di = (sem_di0, sem_di1)
    sem_as = (sem_as0, sem_as1)
    sem_ad = (sem_ad0, sem_ad1)
    sem_xp = (sem_xp0, sem_xp1)
    sem_sc = (sem_sc0, sem_sc1)
    c = lax.axis_index("c")
    s = lax.axis_index("s")
    wid = s * NC + c
    pltpu.sync_copy(zeros_hbm.at[pl.ds(s * RPS, RPS)],
                    acc.at[pl.ds(s * RPS, RPS)])
    pltpu.sync_copy(gmax_hbm, gbuf)
    plsc.subcore_barrier()

    lane = lax.iota(jnp.int32, LANES)
    mask8 = lane < HEADS
    shift_idx = jnp.minimum(lane + HEADS, LANES - 1)
    base0 = wid * EW
    gmaxv = gbuf[...]

    def issue_idx(b, ci):
        base = base0 + ci * CHUNK
        pltpu.async_copy(src_hbm.at[pl.ds(base, CHUNK)], srcv[b], sem_si[b])
        pltpu.async_copy(dst_hbm.at[pl.ds(base, CHUNK)], dstv[b], sem_di[b])

    def wait_idx(b):
        pltpu.make_async_copy(src_hbm.at[pl.ds(0, CHUNK)], srcv[b], sem_si[b]).wait()
        pltpu.make_async_copy(dst_hbm.at[pl.ds(0, CHUNK)], dstv[b], sem_di[b]).wait()

    def issue_gathers(b):
        pltpu.async_copy(atab_hbm.at[srcv[b]], asg[b], sem_as[b])
        pltpu.async_copy(atab_hbm.at[dstv[b]], adg[b], sem_ad[b])
        pltpu.async_copy(xp_hbm.at[srcv[b]], xpg[b], sem_xp[b])

    def wait_gathers(b):
        pltpu.make_async_copy(atab_hbm.at[srcv[b]], asg[b], sem_as[b]).wait()
        pltpu.make_async_copy(atab_hbm.at[dstv[b]], adg[b], sem_ad[b]).wait()
        pltpu.make_async_copy(xp_hbm.at[srcv[b]], xpg[b], sem_xp[b]).wait()

    def wait_scatter(b):
        pltpu.make_async_copy(msg[b], acc.at[dstv2[b]], sem_sc[b]).wait()

    def compute(b):
        asg_b, adg_b, xpg_b, msg_b = asg[b], adg[b], xpg[b], msg[b]

        def edge_body(e, carry2):
            a1 = asg_b[e, :]
            a2 = _vgather(adg_b[e, :], shift_idx)
            al = a1 + a2
            al = jnp.maximum(al, 0.2 * al)
            ea = jnp.exp(al - gmaxv)
            ea = jnp.where(mask8, ea, 0.0)
            msg_b[e, pl.ds(HID, LANES)] = ea
            for hh in range(HEADS):
                sp = _vgather(ea, lane * 0 + hh)
                msg_b[e, pl.ds(hh * LANES, LANES)] = (
                    xpg_b[e, pl.ds(hh * LANES, LANES)] * sp)
            return carry2

        lax.fori_loop(0, CHUNK, edge_body, 0)

    def process(b, ci, first):
        wait_gathers(b)
        if not first:
            wait_scatter(b)
        for k in range(CHUNK // LANES):
            dstv2[b][pl.ds(k * LANES, LANES)] = dstv[b][pl.ds(k * LANES, LANES)]
        cn = jnp.minimum(ci + 2, NCHUNK - 1)
        issue_idx(b, cn)
        compute(b)
        pltpu.async_copy(msg[b], acc.at[dstv2[b]], sem_sc[b], add=True)
        wait_idx(b)
        issue_gathers(b)

    for b in (0, 1):
        issue_idx(b, b)
        wait_idx(b)
        issue_gathers(b)
    for b in (0, 1):
        process(b, jnp.int32(b), True)

    def steady(g, carry):
        process(0, 2 * g, False)
        process(1, 2 * g + 1, False)
        return carry

    lax.fori_loop(1, NCHUNK // 2, steady, 0)
    for b in (0, 1):
        wait_gathers(b)
        wait_scatter(b)
    plsc.subcore_barrier()
    pltpu.sync_copy(acc.at[pl.ds(s * RPS, RPS)],
                    out_hbm.at[c, pl.ds(s * RPS, RPS)])


_sc_edge = pl.kernel(
    _sc_edge_body,
    out_type=jax.ShapeDtypeStruct((NC, NPAD, ACC_W), jnp.float32),
    mesh=plsc.VectorSubcoreMesh(core_axis_name="c", subcore_axis_name="s"),
    compiler_params=pltpu.CompilerParams(use_tc_tiling_on_sc=False),
    scratch_types=(
        [pltpu.VMEM((CHUNK,), jnp.int32)] * 6
        + [pltpu.VMEM((CHUNK, LANES), jnp.float32)] * 4
        + [pltpu.VMEM((CHUNK, HID), jnp.float32)] * 2
        + [pltpu.VMEM((CHUNK, ACC_W), jnp.float32)] * 2
        + [pltpu.VMEM((LANES,), jnp.float32)]
        + [pltpu.VMEM_SHARED((NPAD, ACC_W), jnp.float32)]
        + [pltpu.SemaphoreType.DMA] * 12
    ),
)


def _edge_phase(xp, asn, adn, gmax, src_p, dst_p, zeros_tab):
    atab = jnp.concatenate([asn, adn], axis=1)
    atab = jnp.pad(atab, ((0, NPAD - N), (0, 0)), constant_values=-1e30)
    xp_p = jnp.pad(xp, ((0, NPAD - N), (0, 0)))
    gmax16 = jnp.pad(gmax, ((0, 0), (0, LANES - HEADS))).reshape(LANES)
    return _sc_edge(src_p, dst_p, atab, xp_p, gmax16, zeros_tab)


def kernel(x, edge_index, batch, dop_levels, params):
    p = params
    loops = jnp.arange(N, dtype=edge_index.dtype)
    pad = jnp.full((EP - E - N,), N, dtype=edge_index.dtype)
    src_p = jnp.concatenate([edge_index[0], loops, pad])
    dst_p = jnp.concatenate([edge_index[1], loops, pad])
    zeros_tab = jnp.zeros((NPAD, ACC_W), jnp.float32)

    def asel(a):
        return jax.scipy.linalg.block_diag(*[a[h][:, None] for h in range(HEADS)])

    sexp = jax.scipy.linalg.block_diag(*([jnp.ones((1, CPH), jnp.float32)] * HEADS))
    mavg = jnp.tile(jnp.eye(CPH, dtype=jnp.float32), (HEADS, 1)) / HEADS

    h, xp, asn, adn, gmax = _tc_in(
        x, p['in_W'], p['in_b'].reshape(1, -1), p['g0_W'],
        asel(p['g0_as']), asel(p['g0_ad']))

    for i in (0, 1, 2):
        acc = _edge_phase(xp, asn, adn, gmax, src_p, dst_p, zeros_tab)
        if i < 2:
            j = i + 1
            h, xp, asn, adn, gmax = _tc_mid(
                acc, sexp, p['g%d_b' % i].reshape(1, -1), h, p['g%d_W' % j],
                asel(p['g%d_as' % j]), asel(p['g%d_ad' % j]))
        else:
            h2 = _tc_last(acc, sexp, mavg, p['g2_b'].reshape(1, -1))

    lat, emb = _tc_head(
        h2, batch.reshape(N, 1), dop_levels.reshape(B, 1),
        p['pool_W'], p['pool_b'].reshape(1, -1),
        p['emb_W'], p['emb_b'].reshape(1, -1),
        jnp.pad(p['p0_W'], ((0, 7), (0, 0))), p['p0_b'].reshape(1, -1),
        p['p1_W'], p['p1_b'].reshape(1, -1),
        p['p2_W'], p['p2_b'].reshape(1, -1),
        p['head_W'], p['head_b'].reshape(1, -1))
    return lat, emb
